# baseline (speedup 1.0000x reference)
import jax
import jax.numpy as jnp
from jax.experimental import pallas as pl

_BATCH = 128
_NUM_QUERIES = 1024
_QUERIES_DIM = 512
_B_BLK = 1


def _broadcast_body(w_ref, o_ref):
    o_ref[...] = jnp.broadcast_to(w_ref[...][None], o_ref.shape)


def kernel(queries_weight, batch_size, num_queries):
    del batch_size, num_queries
    return pl.pallas_call(
        _broadcast_body,
        grid=(_BATCH // _B_BLK,),
        in_specs=[
            pl.BlockSpec((_NUM_QUERIES, _QUERIES_DIM), lambda i: (0, 0)),
        ],
        out_specs=pl.BlockSpec(
            (_B_BLK, _NUM_QUERIES, _QUERIES_DIM), lambda i: (i, 0, 0)
        ),
        out_shape=jax.ShapeDtypeStruct(
            (_BATCH, _NUM_QUERIES, _QUERIES_DIM), queries_weight.dtype
        ),
    )(queries_weight)


# speedup vs baseline: 1.1524x; 1.1524x over previous
import jax
import jax.numpy as jnp
from jax.experimental import pallas as pl
from jax.experimental.pallas import tpu as pltpu

_BATCH = 128
_NUM_QUERIES = 1024
_QUERIES_DIM = 512
_NSEM = 8


def _body(w_hbm, o_hbm, w_vmem, in_sem, out_sems):
    load = pltpu.make_async_copy(w_hbm, w_vmem, in_sem)
    load.start()
    load.wait()
    for b in range(_BATCH):
        if b >= _NSEM:
            pltpu.make_async_copy(
                w_vmem, o_hbm.at[b - _NSEM], out_sems.at[(b - _NSEM) % _NSEM]
            ).wait()
        pltpu.make_async_copy(w_vmem, o_hbm.at[b], out_sems.at[b % _NSEM]).start()
    for b in range(_BATCH - _NSEM, _BATCH):
        pltpu.make_async_copy(w_vmem, o_hbm.at[b], out_sems.at[b % _NSEM]).wait()


def kernel(queries_weight, batch_size, num_queries):
    del batch_size, num_queries
    return pl.pallas_call(
        _body,
        in_specs=[pl.BlockSpec(memory_space=pltpu.MemorySpace.HBM)],
        out_specs=pl.BlockSpec(memory_space=pltpu.MemorySpace.HBM),
        out_shape=jax.ShapeDtypeStruct(
            (_BATCH, _NUM_QUERIES, _QUERIES_DIM), queries_weight.dtype
        ),
        scratch_shapes=[
            pltpu.VMEM((_NUM_QUERIES, _QUERIES_DIM), jnp.float32),
            pltpu.SemaphoreType.DMA,
            pltpu.SemaphoreType.DMA((_NSEM,)),
        ],
    )(queries_weight)
